# baseline (speedup 1.0000x reference)
import functools

import jax
import jax.numpy as jnp
from jax import lax
from jax.experimental import pallas as pl
from jax.experimental.pallas import tpu as pltpu
from jax.experimental.pallas import tpu_sc as plsc

_NUM_CORES = 2
_NUM_SUBCORES = 16
_NUM_WORKERS = _NUM_CORES * _NUM_SUBCORES
_NBUF = 4


@functools.lru_cache(maxsize=None)
def _make_gather_kernel(num_rows: int, dim: int, chunk: int):
    rows_per_worker = num_rows // _NUM_WORKERS
    num_chunks = rows_per_worker // chunk
    assert num_rows % _NUM_WORKERS == 0
    assert rows_per_worker % chunk == 0
    assert num_chunks % _NBUF == 0 and num_chunks >= 2 * _NBUF
    assert chunk % 8 == 0

    mesh = plsc.VectorSubcoreMesh(
        core_axis_name="c",
        subcore_axis_name="s",
        num_cores=_NUM_CORES,
        num_subcores=_NUM_SUBCORES,
    )

    @functools.partial(
        pl.kernel,
        mesh=mesh,
        out_type=jax.ShapeDtypeStruct((num_rows, dim), jnp.float32),
        scratch_types=[
            pltpu.VMEM((rows_per_worker,), jnp.int32),
            [pltpu.VMEM((chunk, dim), jnp.float32) for _ in range(_NBUF)],
            [pltpu.SemaphoreType.DMA for _ in range(_NBUF)],
            [pltpu.SemaphoreType.DMA for _ in range(_NBUF)],
        ],
    )
    def gather_kernel(table_hbm, idx_hbm, out_hbm, idx_v, bufs, gsems, ssems):
        wid = lax.axis_index("s") * _NUM_CORES + lax.axis_index("c")
        base = wid * rows_per_worker
        pltpu.sync_copy(idx_hbm.at[pl.ds(base, rows_per_worker)], idx_v)

        def start_gather(chunk_id, b):
            off = chunk_id * chunk
            pltpu.async_copy(
                table_hbm.at[idx_v.at[pl.ds(off, chunk)]], bufs[b], gsems[b]
            )

        def wait_gather(b):
            pltpu.make_async_copy(
                table_hbm.at[pl.ds(0, chunk)], bufs[b], gsems[b]
            ).wait()

        def start_store(chunk_id, b):
            pltpu.async_copy(
                bufs[b], out_hbm.at[pl.ds(base + chunk_id * chunk, chunk)], ssems[b]
            )

        def wait_store(b):
            pltpu.make_async_copy(
                bufs[b], out_hbm.at[pl.ds(base, chunk)], ssems[b]
            ).wait()

        start_gather(0, 0)
        start_gather(1, 1)
        wait_gather(0)
        start_store(0, 0)
        start_gather(2, 2)
        wait_gather(1)
        start_store(1, 1)
        start_gather(3, 3)

        def body(g, carry):
            for b in range(_NBUF):
                i = g * _NBUF + b
                wait_store(b)
                start_gather(i, b)
                b2 = (b + 2) % _NBUF
                wait_gather(b2)
                start_store(i - 2, b2)
            return carry

        lax.fori_loop(1, num_chunks // _NBUF, body, 0, unroll=False)

        n = num_chunks
        wait_gather((n - 2) % _NBUF)
        start_store(n - 2, (n - 2) % _NBUF)
        wait_gather((n - 1) % _NBUF)
        start_store(n - 1, (n - 1) % _NBUF)
        for b in range(_NBUF):
            wait_store(b)

    return gather_kernel


def kernel(token_ids, weight):
    dim = weight.shape[1]
    idx = token_ids.reshape(-1).astype(jnp.int32)
    gather = _make_gather_kernel(idx.shape[0], dim, 200)
    out = gather(weight, idx)
    return out.reshape(token_ids.shape + (dim,))
